# Initial kernel scaffold; baseline (speedup 1.0000x reference)
import jax
import jax.numpy as jnp
from jax.experimental import pallas as pl


def kernel(x, trafo):
    raise NotImplementedError("write your pallas kernel here")


# speedup vs baseline: 2.0709x; 2.0709x over previous
import functools

import jax
import jax.numpy as jnp
from jax import lax
from jax.experimental import pallas as pl
from jax.experimental.pallas import tpu as pltpu
from jax.experimental.pallas import tpu_sc as plsc

B = 64
N = 4096
P = 128
K = 2048
C = 2
OG = K // C

NC, NS = 2, 16
TOT = B * K
BPC = B // NC
SR = N // NS
GR = K // NS

MB = 1024


_NSTAGE = sum(range(1, 13))


def _topk_sort_body(jj_ref, kk_ref, val_ref, idx_out_ref, key_s, idx_s):
    s = pl.program_id(0)
    row = lax.broadcasted_iota(jnp.int32, (N, B), 0)

    @pl.when(s == 0)
    def _():
        key_s[...] = val_ref[...]
        idx_s[...] = row

    j = jj_ref[s]
    k = kk_ref[s]
    key = key_s[...]
    idx = idx_s[...]
    fwd_key = pltpu.roll(key, N - j, axis=0)
    bwd_key = pltpu.roll(key, j, axis=0)
    fwd_idx = pltpu.roll(idx, N - j, axis=0)
    bwd_idx = pltpu.roll(idx, j, axis=0)
    lower = (row & j) == 0
    pkey = jnp.where(lower, fwd_key, bwd_key)
    pidx = jnp.where(lower, fwd_idx, bwd_idx)
    dirf = (row & k) == 0
    pre = (key > pkey) | ((key == pkey) & (idx < pidx))
    keep = pre == (lower == dirf)
    new_idx = jnp.where(keep, idx, pidx)
    key_s[...] = jnp.where(keep, key, pkey)
    idx_s[...] = new_idx

    @pl.when(s == _NSTAGE - 1)
    def _():
        idx_out_ref[...] = new_idx[:K, :]


_topk_sort = pl.pallas_call(
    _topk_sort_body,
    grid_spec=pltpu.PrefetchScalarGridSpec(
        num_scalar_prefetch=2,
        grid=(_NSTAGE,),
        in_specs=[pl.BlockSpec((N, B), lambda s, jj, kk: (0, 0))],
        out_specs=pl.BlockSpec((K, B), lambda s, jj, kk: (0, 0)),
        scratch_shapes=[
            pltpu.VMEM((N, B), jnp.float32),
            pltpu.VMEM((N, B), jnp.int32),
        ],
    ),
    out_shape=jax.ShapeDtypeStruct((K, B), jnp.int32),
)


def _stage_params():
    jjs, kks = [], []
    k = 2
    while k <= N:
        j = k // 2
        while j >= 1:
            jjs.append(j)
            kks.append(k)
            j //= 2
        k *= 2
    return (jnp.asarray(jjs, jnp.int32), jnp.asarray(kks, jnp.int32))


def _gather_body(x_hbm, idx_hbm, out_hbm, idx_v, rows_v, sp0, sp1,
                 sem_g, st_sem0, st_sem1):
    cid = lax.axis_index("c")
    sid = lax.axis_index("s")
    pltpu.sync_copy(idx_hbm.at[sid, pl.ds(cid * BPC, BPC)], idx_v)
    sps = [sp0, sp1]
    st_sems = [st_sem0, st_sem1]
    st_cps = [None, None]

    def start_stage(step):
        b = cid * BPC + step
        buf = step % 2
        st_cps[buf] = pltpu.async_copy(
            x_hbm.at[pl.ds(b * N + sid * SR, SR)],
            sps[buf].at[pl.ds(sid * SR, SR)],
            st_sems[buf])

    start_stage(0)
    start_stage(1)
    for step in range(BPC):
        buf = step % 2
        st_cps[buf].wait()
        plsc.subcore_barrier()
        pltpu.async_copy(sps[buf].at[idx_v.at[step]], rows_v, sem_g).wait()
        b = cid * BPC + step
        pltpu.sync_copy(rows_v, out_hbm.at[pl.ds(b * K + sid * GR, GR)])
        plsc.subcore_barrier()
        if step + 2 < BPC:
            start_stage(step + 2)


@functools.lru_cache(maxsize=None)
def _build_gather():
    return functools.partial(
        pl.kernel,
        out_type=jax.ShapeDtypeStruct((TOT, P), jnp.float32),
        mesh=plsc.VectorSubcoreMesh(core_axis_name="c", subcore_axis_name="s"),
        scratch_types=[
            pltpu.VMEM((BPC, GR), jnp.int32),
            pltpu.VMEM((GR, P), jnp.float32),
            pltpu.VMEM_SHARED((N, P), jnp.float32),
            pltpu.VMEM_SHARED((N, P), jnp.float32),
            pltpu.SemaphoreType.DMA,
            pltpu.SemaphoreType.DMA,
            pltpu.SemaphoreType.DMA,
        ],
    )(_gather_body)


def _mm_body(xs_ref, w_ref, o_ref):
    o_ref[...] = jnp.dot(xs_ref[...], w_ref[...],
                         preferred_element_type=jnp.float32)


_mm = pl.pallas_call(
    _mm_body,
    grid=(TOT // C // MB,),
    in_specs=[
        pl.BlockSpec((MB, P * C), lambda i: (i, 0)),
        pl.BlockSpec((P * C, P), lambda i: (0, 0)),
    ],
    out_specs=pl.BlockSpec((MB, P), lambda i: (i, 0)),
    out_shape=jax.ShapeDtypeStruct((TOT // C, P), jnp.float32),
)


def kernel(x, trafo):
    x2d = x.reshape(B * N, P)
    values_t = x[:, :, P - 1].T
    jjs, kks = _stage_params()
    order_t = _topk_sort(jjs, kks, values_t)
    order = order_t.T.reshape(B, NS, GR).transpose(1, 0, 2)
    xg = _build_gather()(x2d, order)
    xs = xg.reshape(TOT // C, P * C)
    traf = _mm(xs, trafo)
    return traf.reshape(B, OG, P)


# speedup vs baseline: 3.9681x; 1.9161x over previous
import functools

import jax
import jax.numpy as jnp
from jax import lax
from jax.experimental import pallas as pl
from jax.experimental.pallas import tpu as pltpu
from jax.experimental.pallas import tpu_sc as plsc

B = 64
N = 4096
P = 128
K = 2048
C = 2
OG = K // C

NC, NS = 2, 16
TOT = B * K
BPC = B // NC
SR = N // NS
GR = K // NS

MB = 1024


_NSTAGE = sum(range(1, 13))


def _partner(arr, jm):
    if jm >= 8:
        a3 = arr.reshape(N // (2 * jm), 2, jm, B)
        sw = jnp.concatenate([a3[:, 1:2], a3[:, 0:1]], axis=1)
        return sw.reshape(N, B)
    fwd = jnp.concatenate([arr[jm:], arr[:jm]], axis=0)
    bwd = jnp.concatenate([arr[N - jm:], arr[:N - jm]], axis=0)
    row = lax.broadcasted_iota(jnp.int32, (N, B), 0)
    return jnp.where((row & jm) == 0, fwd, bwd)


def _topk_sort_body(jj_ref, kk_ref, val_ref, idx_out_ref,
                    key_s, idx_s, pk_s, pi_s):
    s = pl.program_id(0)
    row = lax.broadcasted_iota(jnp.int32, (N, B), 0)

    @pl.when(s == 0)
    def _():
        key_s[...] = val_ref[...]
        idx_s[...] = row

    j = jj_ref[s]
    k = kk_ref[s]
    key = key_s[...]
    idx = idx_s[...]
    for m in range(12):
        jm = 2 ** m

        @pl.when(j == jm)
        def _(jm=jm):
            pk_s[...] = _partner(key, jm)
            pi_s[...] = _partner(idx, jm)

    pkey = pk_s[...]
    pidx = pi_s[...]
    lower = (row & j) == 0
    dirf = (row & k) == 0
    pre = (key > pkey) | ((key == pkey) & (idx < pidx))
    keep = pre == (lower == dirf)
    new_idx = jnp.where(keep, idx, pidx)
    key_s[...] = jnp.where(keep, key, pkey)
    idx_s[...] = new_idx

    @pl.when(s == _NSTAGE - 1)
    def _():
        idx_out_ref[...] = new_idx[:K, :]


_topk_sort = pl.pallas_call(
    _topk_sort_body,
    grid_spec=pltpu.PrefetchScalarGridSpec(
        num_scalar_prefetch=2,
        grid=(_NSTAGE,),
        in_specs=[pl.BlockSpec((N, B), lambda s, jj, kk: (0, 0))],
        out_specs=pl.BlockSpec((K, B), lambda s, jj, kk: (0, 0)),
        scratch_shapes=[
            pltpu.VMEM((N, B), jnp.float32),
            pltpu.VMEM((N, B), jnp.int32),
            pltpu.VMEM((N, B), jnp.float32),
            pltpu.VMEM((N, B), jnp.int32),
        ],
    ),
    out_shape=jax.ShapeDtypeStruct((K, B), jnp.int32),
)


def _stage_params():
    jjs, kks = [], []
    k = 2
    while k <= N:
        j = k // 2
        while j >= 1:
            jjs.append(j)
            kks.append(k)
            j //= 2
        k *= 2
    return (jnp.asarray(jjs, jnp.int32), jnp.asarray(kks, jnp.int32))


def _gather_body(x_hbm, idx_hbm, out_hbm, idx_v, rows_v, sp0, sp1,
                 sem_g, st_sem0, st_sem1):
    cid = lax.axis_index("c")
    sid = lax.axis_index("s")
    pltpu.sync_copy(idx_hbm.at[sid, pl.ds(cid * BPC, BPC)], idx_v)
    sps = [sp0, sp1]
    st_sems = [st_sem0, st_sem1]
    st_cps = [None, None]

    def start_stage(step):
        b = cid * BPC + step
        buf = step % 2
        st_cps[buf] = pltpu.async_copy(
            x_hbm.at[pl.ds(b * N + sid * SR, SR)],
            sps[buf].at[pl.ds(sid * SR, SR)],
            st_sems[buf])

    start_stage(0)
    start_stage(1)
    for step in range(BPC):
        buf = step % 2
        st_cps[buf].wait()
        plsc.subcore_barrier()
        pltpu.async_copy(sps[buf].at[idx_v.at[step]], rows_v, sem_g).wait()
        b = cid * BPC + step
        pltpu.sync_copy(rows_v, out_hbm.at[pl.ds(b * K + sid * GR, GR)])
        plsc.subcore_barrier()
        if step + 2 < BPC:
            start_stage(step + 2)


@functools.lru_cache(maxsize=None)
def _build_gather():
    return functools.partial(
        pl.kernel,
        out_type=jax.ShapeDtypeStruct((TOT, P), jnp.float32),
        mesh=plsc.VectorSubcoreMesh(core_axis_name="c", subcore_axis_name="s"),
        scratch_types=[
            pltpu.VMEM((BPC, GR), jnp.int32),
            pltpu.VMEM((GR, P), jnp.float32),
            pltpu.VMEM_SHARED((N, P), jnp.float32),
            pltpu.VMEM_SHARED((N, P), jnp.float32),
            pltpu.SemaphoreType.DMA,
            pltpu.SemaphoreType.DMA,
            pltpu.SemaphoreType.DMA,
        ],
    )(_gather_body)


def _mm_body(xs_ref, w_ref, o_ref):
    o_ref[...] = jnp.dot(xs_ref[...], w_ref[...],
                         preferred_element_type=jnp.float32)


_mm = pl.pallas_call(
    _mm_body,
    grid=(TOT // C // MB,),
    in_specs=[
        pl.BlockSpec((MB, P * C), lambda i: (i, 0)),
        pl.BlockSpec((P * C, P), lambda i: (0, 0)),
    ],
    out_specs=pl.BlockSpec((MB, P), lambda i: (i, 0)),
    out_shape=jax.ShapeDtypeStruct((TOT // C, P), jnp.float32),
)


def kernel(x, trafo):
    x2d = x.reshape(B * N, P)
    values_t = x[:, :, P - 1].T
    jjs, kks = _stage_params()
    order_t = _topk_sort(jjs, kks, values_t)
    order = order_t.T.reshape(B, NS, GR).transpose(1, 0, 2)
    xg = _build_gather()(x2d, order)
    xs = xg.reshape(TOT // C, P * C)
    traf = _mm(xs, trafo)
    return traf.reshape(B, OG, P)


# speedup vs baseline: 5.6742x; 1.4300x over previous
import functools

import jax
import jax.numpy as jnp
from jax import lax
from jax.experimental import pallas as pl
from jax.experimental.pallas import tpu as pltpu
from jax.experimental.pallas import tpu_sc as plsc

B = 64
N = 4096
P = 128
K = 2048
C = 2
OG = K // C

NC, NS = 2, 16
TOT = B * K
BPC = B // NC
SR = N // NS
GR = K // NS
GH = GR // 2

MB = 1024


_NSTAGE = sum(range(1, 13))


def _partner(arr, jm):
    if jm >= 8:
        a3 = arr.reshape(N // (2 * jm), 2, jm, B)
        sw = jnp.concatenate([a3[:, 1:2], a3[:, 0:1]], axis=1)
        return sw.reshape(N, B)
    fwd = jnp.concatenate([arr[jm:], arr[:jm]], axis=0)
    bwd = jnp.concatenate([arr[N - jm:], arr[:N - jm]], axis=0)
    row = lax.broadcasted_iota(jnp.int32, (N, B), 0)
    return jnp.where((row & jm) == 0, fwd, bwd)


def _topk_sort_body(jj_ref, kk_ref, val_ref, idx_out_ref,
                    key_s, idx_s, pk_s, pi_s):
    s = pl.program_id(0)
    row = lax.broadcasted_iota(jnp.int32, (N, B), 0)

    @pl.when(s == 0)
    def _():
        key_s[...] = val_ref[...]
        idx_s[...] = row

    j = jj_ref[s]
    k = kk_ref[s]
    key = key_s[...]
    idx = idx_s[...]
    for m in range(12):
        jm = 2 ** m

        @pl.when(j == jm)
        def _(jm=jm):
            pk_s[...] = _partner(key, jm)
            pi_s[...] = _partner(idx, jm)

    pkey = pk_s[...]
    pidx = pi_s[...]
    lower = (row & j) == 0
    dirf = (row & k) == 0
    pre = (key > pkey) | ((key == pkey) & (idx < pidx))
    keep = pre == (lower == dirf)
    new_idx = jnp.where(keep, idx, pidx)
    key_s[...] = jnp.where(keep, key, pkey)
    idx_s[...] = new_idx

    @pl.when(s == _NSTAGE - 1)
    def _():
        idx_out_ref[...] = new_idx[:K, :]


_topk_sort = pl.pallas_call(
    _topk_sort_body,
    grid_spec=pltpu.PrefetchScalarGridSpec(
        num_scalar_prefetch=2,
        grid=(_NSTAGE,),
        in_specs=[pl.BlockSpec((N, B), lambda s, jj, kk: (0, 0))],
        out_specs=pl.BlockSpec((K, B), lambda s, jj, kk: (0, 0)),
        scratch_shapes=[
            pltpu.VMEM((N, B), jnp.float32),
            pltpu.VMEM((N, B), jnp.int32),
            pltpu.VMEM((N, B), jnp.float32),
            pltpu.VMEM((N, B), jnp.int32),
        ],
    ),
    out_shape=jax.ShapeDtypeStruct((K, B), jnp.int32),
)


def _stage_params():
    jjs, kks = [], []
    k = 2
    while k <= N:
        j = k // 2
        while j >= 1:
            jjs.append(j)
            kks.append(k)
            j //= 2
        k *= 2
    return (jnp.asarray(jjs, jnp.int32), jnp.asarray(kks, jnp.int32))


_VC = N // P


def _extract_body(x1_hbm, idx_hbm, out_hbm, idx_v, vals_v, sem_g):
    cid = lax.axis_index("c")
    sid = lax.axis_index("s")
    wid = sid * NC + cid
    for rel in range(B // (NC * NS)):
        b = wid * (B // (NC * NS)) + rel
        pltpu.sync_copy(idx_hbm.at[b], idx_v)
        cps = [pltpu.async_copy(x1_hbm.at[idx_v.at[c]], vals_v.at[c], sem_g)
               for c in range(_VC)]
        for cp in cps:
            cp.wait()
        pltpu.sync_copy(vals_v, out_hbm.at[b])


@functools.lru_cache(maxsize=None)
def _build_extract():
    return functools.partial(
        pl.kernel,
        out_type=jax.ShapeDtypeStruct((B, _VC, P), jnp.float32),
        mesh=plsc.VectorSubcoreMesh(core_axis_name="c", subcore_axis_name="s"),
        scratch_types=[
            pltpu.VMEM((_VC, P), jnp.int32),
            pltpu.VMEM((_VC, P), jnp.float32),
            pltpu.SemaphoreType.DMA,
        ],
    )(_extract_body)


def _gather_body(x_hbm, idx_hbm, out_hbm, idx_v, rows_e, rows_o, sp0, sp1,
                 sem_g, st_sem0, st_sem1):
    cid = lax.axis_index("c")
    sid = lax.axis_index("s")
    pltpu.sync_copy(idx_hbm.at[sid, pl.ds(cid * BPC, BPC)], idx_v)
    sps = [sp0, sp1]
    st_sems = [st_sem0, st_sem1]
    st_cps = [None, None]

    def start_stage(step):
        b = cid * BPC + step
        buf = step % 2
        st_cps[buf] = pltpu.async_copy(
            x_hbm.at[pl.ds(b * N + sid * SR, SR)],
            sps[buf].at[pl.ds(sid * SR, SR)],
            st_sems[buf])

    start_stage(0)
    start_stage(1)
    for step in range(BPC):
        buf = step % 2
        st_cps[buf].wait()
        plsc.subcore_barrier()
        cp_e = pltpu.async_copy(sps[buf].at[idx_v.at[step, 0]], rows_e, sem_g)
        cp_o = pltpu.async_copy(sps[buf].at[idx_v.at[step, 1]], rows_o, sem_g)
        cp_e.wait()
        cp_o.wait()
        b = cid * BPC + step
        gbase = b * OG + sid * GH
        pltpu.sync_copy(rows_e, out_hbm.at[pl.ds(gbase, GH), pl.ds(0, P)])
        pltpu.sync_copy(rows_o, out_hbm.at[pl.ds(gbase, GH), pl.ds(P, P)])
        plsc.subcore_barrier()
        if step + 2 < BPC:
            start_stage(step + 2)


@functools.lru_cache(maxsize=None)
def _build_gather():
    return functools.partial(
        pl.kernel,
        out_type=jax.ShapeDtypeStruct((TOT // C, C * P), jnp.float32),
        mesh=plsc.VectorSubcoreMesh(core_axis_name="c", subcore_axis_name="s"),
        scratch_types=[
            pltpu.VMEM((BPC, 2, GH), jnp.int32),
            pltpu.VMEM((GH, P), jnp.float32),
            pltpu.VMEM((GH, P), jnp.float32),
            pltpu.VMEM_SHARED((N, P), jnp.float32),
            pltpu.VMEM_SHARED((N, P), jnp.float32),
            pltpu.SemaphoreType.DMA,
            pltpu.SemaphoreType.DMA,
            pltpu.SemaphoreType.DMA,
        ],
    )(_gather_body)


def _mm_body(xs_ref, w_ref, o_ref):
    o_ref[...] = jnp.dot(xs_ref[...], w_ref[...],
                         preferred_element_type=jnp.float32)


_mm = pl.pallas_call(
    _mm_body,
    grid=(TOT // C // MB,),
    in_specs=[
        pl.BlockSpec((MB, P * C), lambda i: (i, 0)),
        pl.BlockSpec((P * C, P), lambda i: (0, 0)),
    ],
    out_specs=pl.BlockSpec((MB, P), lambda i: (i, 0)),
    out_shape=jax.ShapeDtypeStruct((TOT // C, P), jnp.float32),
)


def kernel(x, trafo):
    x2d = x.reshape(B * N, P)
    eidx = (jnp.arange(B * N, dtype=jnp.int32) * P + (P - 1)).reshape(
        B, _VC, P)
    vals = _build_extract()(x.reshape(-1), eidx)
    values_t = vals.reshape(B, N).T
    jjs, kks = _stage_params()
    order_t = _topk_sort(jjs, kks, values_t)
    order = order_t.T.reshape(B, NS, GH, 2).transpose(1, 0, 3, 2)
    xs = _build_gather()(x2d, order)
    traf = _mm(xs, trafo)
    return traf.reshape(B, OG, P)


# speedup vs baseline: 7.7302x; 1.3623x over previous
import functools

import jax
import jax.numpy as jnp
from jax import lax
from jax.experimental import pallas as pl
from jax.experimental.pallas import tpu as pltpu
from jax.experimental.pallas import tpu_sc as plsc

B = 64
N = 4096
P = 128
K = 2048
C = 2
OG = K // C

NC, NS = 2, 16
TOT = B * K
BPC = B // NC
SR = N // NS
GR = K // NS
GH = GR // 2

MB = 1024


_NSTAGE = sum(range(1, 13))


NH = N // 2


def _partner(arr, jm):
    if jm == NH:
        return pltpu.roll(arr, B, axis=1)
    if jm >= 8:
        a3 = arr.reshape(NH // (2 * jm), 2, jm, 2 * B)
        sw = jnp.concatenate([a3[:, 1:2], a3[:, 0:1]], axis=1)
        return sw.reshape(NH, 2 * B)
    fwd = jnp.concatenate([arr[jm:], arr[:jm]], axis=0)
    bwd = jnp.concatenate([arr[NH - jm:], arr[:NH - jm]], axis=0)
    row = lax.broadcasted_iota(jnp.int32, (NH, 2 * B), 0)
    return jnp.where((row & jm) == 0, fwd, bwd)


def _topk_sort_body(jj_ref, kk_ref, val_ref, idx_out_ref,
                    key_s, idx_s, pk_s, pi_s):
    s = pl.program_id(0)
    row = lax.broadcasted_iota(jnp.int32, (NH, 2 * B), 0)
    lane = lax.broadcasted_iota(jnp.int32, (NH, 2 * B), 1)
    ifull = row + ((lane & B) << 5)

    @pl.when(s == 0)
    def _():
        key_s[...] = val_ref[...]
        idx_s[...] = ifull

    j = jj_ref[s]
    k = kk_ref[s]
    key = key_s[...]
    idx = idx_s[...]
    for m in range(12):
        jm = 2 ** m

        @pl.when(j == jm)
        def _(jm=jm):
            pk_s[...] = _partner(key, jm)
            pi_s[...] = _partner(idx, jm)

    pkey = pk_s[...]
    pidx = pi_s[...]
    lower = (ifull & j) == 0
    dirf = (ifull & k) == 0
    pre = (key > pkey) | ((key == pkey) & (idx < pidx))
    keep = pre == (lower == dirf)
    new_idx = jnp.where(keep, idx, pidx)
    key_s[...] = jnp.where(keep, key, pkey)
    idx_s[...] = new_idx

    @pl.when(s == _NSTAGE - 1)
    def _():
        idx_out_ref[...] = new_idx[:, :B]


_topk_sort = pl.pallas_call(
    _topk_sort_body,
    grid_spec=pltpu.PrefetchScalarGridSpec(
        num_scalar_prefetch=2,
        grid=(_NSTAGE,),
        in_specs=[pl.BlockSpec((NH, 2 * B), lambda s, jj, kk: (0, 0))],
        out_specs=pl.BlockSpec((K, B), lambda s, jj, kk: (0, 0)),
        scratch_shapes=[
            pltpu.VMEM((NH, 2 * B), jnp.float32),
            pltpu.VMEM((NH, 2 * B), jnp.int32),
            pltpu.VMEM((NH, 2 * B), jnp.float32),
            pltpu.VMEM((NH, 2 * B), jnp.int32),
        ],
    ),
    out_shape=jax.ShapeDtypeStruct((K, B), jnp.int32),
)


def _stage_params():
    jjs, kks = [], []
    k = 2
    while k <= N:
        j = k // 2
        while j >= 1:
            jjs.append(j)
            kks.append(k)
            j //= 2
        k *= 2
    return (jnp.asarray(jjs, jnp.int32), jnp.asarray(kks, jnp.int32))


_VC = N // P


def _extract_body(x1_hbm, idx_hbm, out_hbm, idx_v, vals_v, sem_g):
    cid = lax.axis_index("c")
    sid = lax.axis_index("s")
    wid = sid * NC + cid
    for rel in range(B // (NC * NS)):
        b = wid * (B // (NC * NS)) + rel
        pltpu.sync_copy(idx_hbm.at[b], idx_v)
        cps = [pltpu.async_copy(x1_hbm.at[idx_v.at[c]], vals_v.at[c], sem_g)
               for c in range(_VC)]
        for cp in cps:
            cp.wait()
        pltpu.sync_copy(vals_v, out_hbm.at[b])


@functools.lru_cache(maxsize=None)
def _build_extract():
    return functools.partial(
        pl.kernel,
        out_type=jax.ShapeDtypeStruct((B, _VC, P), jnp.float32),
        mesh=plsc.VectorSubcoreMesh(core_axis_name="c", subcore_axis_name="s"),
        scratch_types=[
            pltpu.VMEM((_VC, P), jnp.int32),
            pltpu.VMEM((_VC, P), jnp.float32),
            pltpu.SemaphoreType.DMA,
        ],
    )(_extract_body)


def _gather_body(x_hbm, idx_hbm, out_hbm, idx_v, rows_e, rows_o, sp0, sp1,
                 sem_g, st_sem0, st_sem1):
    cid = lax.axis_index("c")
    sid = lax.axis_index("s")
    pltpu.sync_copy(idx_hbm.at[sid, pl.ds(cid * BPC, BPC)], idx_v)
    sps = [sp0, sp1]
    st_sems = [st_sem0, st_sem1]
    st_cps = [None, None]

    def start_stage(step):
        b = cid * BPC + step
        buf = step % 2
        st_cps[buf] = pltpu.async_copy(
            x_hbm.at[pl.ds(b * N + sid * SR, SR)],
            sps[buf].at[pl.ds(sid * SR, SR)],
            st_sems[buf])

    start_stage(0)
    start_stage(1)
    for step in range(BPC):
        buf = step % 2
        st_cps[buf].wait()
        plsc.subcore_barrier()
        cp_e = pltpu.async_copy(sps[buf].at[idx_v.at[step, 0]], rows_e, sem_g)
        cp_o = pltpu.async_copy(sps[buf].at[idx_v.at[step, 1]], rows_o, sem_g)
        cp_e.wait()
        cp_o.wait()
        b = cid * BPC + step
        gbase = b * OG + sid * GH
        pltpu.sync_copy(rows_e, out_hbm.at[pl.ds(gbase, GH), pl.ds(0, P)])
        pltpu.sync_copy(rows_o, out_hbm.at[pl.ds(gbase, GH), pl.ds(P, P)])
        plsc.subcore_barrier()
        if step + 2 < BPC:
            start_stage(step + 2)


@functools.lru_cache(maxsize=None)
def _build_gather():
    return functools.partial(
        pl.kernel,
        out_type=jax.ShapeDtypeStruct((TOT // C, C * P), jnp.float32),
        mesh=plsc.VectorSubcoreMesh(core_axis_name="c", subcore_axis_name="s"),
        scratch_types=[
            pltpu.VMEM((BPC, 2, GH), jnp.int32),
            pltpu.VMEM((GH, P), jnp.float32),
            pltpu.VMEM((GH, P), jnp.float32),
            pltpu.VMEM_SHARED((N, P), jnp.float32),
            pltpu.VMEM_SHARED((N, P), jnp.float32),
            pltpu.SemaphoreType.DMA,
            pltpu.SemaphoreType.DMA,
            pltpu.SemaphoreType.DMA,
        ],
    )(_gather_body)


def _mm_body(xs_ref, w_ref, o_ref):
    o_ref[...] = jnp.dot(xs_ref[...], w_ref[...],
                         preferred_element_type=jnp.float32)


_mm = pl.pallas_call(
    _mm_body,
    grid=(TOT // C // MB,),
    in_specs=[
        pl.BlockSpec((MB, P * C), lambda i: (i, 0)),
        pl.BlockSpec((P * C, P), lambda i: (0, 0)),
    ],
    out_specs=pl.BlockSpec((MB, P), lambda i: (i, 0)),
    out_shape=jax.ShapeDtypeStruct((TOT // C, P), jnp.float32),
)


def kernel(x, trafo):
    x2d = x.reshape(B * N, P)
    eidx = (jnp.arange(B * N, dtype=jnp.int32) * P + (P - 1)).reshape(
        B, _VC, P)
    vals = _build_extract()(x.reshape(-1), eidx)
    values_f = vals.reshape(B, 2, NH).transpose(2, 1, 0).reshape(NH, 2 * B)
    jjs, kks = _stage_params()
    order_t = _topk_sort(jjs, kks, values_f)
    order = order_t.T.reshape(B, NS, GH, 2).transpose(1, 0, 3, 2)
    xs = _build_gather()(x2d, order)
    traf = _mm(xs, trafo)
    return traf.reshape(B, OG, P)


# speedup vs baseline: 8.1213x; 1.0506x over previous
import functools

import jax
import jax.numpy as jnp
from jax import lax
from jax.experimental import pallas as pl
from jax.experimental.pallas import tpu as pltpu
from jax.experimental.pallas import tpu_sc as plsc

B = 64
N = 4096
P = 128
K = 2048
C = 2
OG = K // C

NC, NS = 2, 16
TOT = B * K
BPC = B // NC
SR = N // NS
GR = K // NS
GH = GR // 2

MB = 1024


_NSTAGE = sum(range(1, 13))


NH = N // 2


def _partner(arr, jm):
    if jm == NH:
        return pltpu.roll(arr, B, axis=1)
    if jm >= 8:
        a3 = arr.reshape(NH // (2 * jm), 2, jm, 2 * B)
        sw = jnp.concatenate([a3[:, 1:2], a3[:, 0:1]], axis=1)
        return sw.reshape(NH, 2 * B)
    fwd = jnp.concatenate([arr[jm:], arr[:jm]], axis=0)
    bwd = jnp.concatenate([arr[NH - jm:], arr[:NH - jm]], axis=0)
    row = lax.broadcasted_iota(jnp.int32, (NH, 2 * B), 0)
    return jnp.where((row & jm) == 0, fwd, bwd)


def _topk_sort_body(jj_ref, kk_ref, val_ref, idx_out_ref, key_s, idx_s):
    s = pl.program_id(0)
    row = lax.broadcasted_iota(jnp.int32, (NH, 2 * B), 0)
    lane = lax.broadcasted_iota(jnp.int32, (NH, 2 * B), 1)
    ifull = row + ((lane & B) << 5)

    @pl.when(s == 0)
    def _():
        key_s[...] = val_ref[...]
        idx_s[...] = ifull

    j = jj_ref[s]
    k = kk_ref[s]
    key = key_s[...]
    idx = idx_s[...]
    dirf = (ifull & k) == 0
    for m in range(12):
        jm = 2 ** m

        @pl.when(j == jm)
        def _(jm=jm):
            pkey = _partner(key, jm)
            pidx = _partner(idx, jm)
            lower = (ifull & jm) == 0
            pre = (key > pkey) | ((key == pkey) & (idx < pidx))
            keep = pre == (lower == dirf)
            key_s[...] = jnp.where(keep, key, pkey)
            idx_s[...] = jnp.where(keep, idx, pidx)

    @pl.when(s == _NSTAGE - 1)
    def _():
        idx_out_ref[...] = idx_s[:K, :B]


_topk_sort = pl.pallas_call(
    _topk_sort_body,
    grid_spec=pltpu.PrefetchScalarGridSpec(
        num_scalar_prefetch=2,
        grid=(_NSTAGE,),
        in_specs=[pl.BlockSpec((NH, 2 * B), lambda s, jj, kk: (0, 0))],
        out_specs=pl.BlockSpec((K, B), lambda s, jj, kk: (0, 0)),
        scratch_shapes=[
            pltpu.VMEM((NH, 2 * B), jnp.float32),
            pltpu.VMEM((NH, 2 * B), jnp.int32),
        ],
    ),
    out_shape=jax.ShapeDtypeStruct((K, B), jnp.int32),
)


def _stage_params():
    jjs, kks = [], []
    k = 2
    while k <= N:
        j = k // 2
        while j >= 1:
            jjs.append(j)
            kks.append(k)
            j //= 2
        k *= 2
    return (jnp.asarray(jjs, jnp.int32), jnp.asarray(kks, jnp.int32))


_VC = N // P


def _extract_body(x1_hbm, idx_hbm, out_hbm, idx_v, vals_v, sem_g):
    cid = lax.axis_index("c")
    sid = lax.axis_index("s")
    wid = sid * NC + cid
    for rel in range(B // (NC * NS)):
        b = wid * (B // (NC * NS)) + rel
        pltpu.sync_copy(idx_hbm.at[b], idx_v)
        cps = [pltpu.async_copy(x1_hbm.at[idx_v.at[c]], vals_v.at[c], sem_g)
               for c in range(_VC)]
        for cp in cps:
            cp.wait()
        pltpu.sync_copy(vals_v, out_hbm.at[b])


@functools.lru_cache(maxsize=None)
def _build_extract():
    return functools.partial(
        pl.kernel,
        out_type=jax.ShapeDtypeStruct((B, _VC, P), jnp.float32),
        mesh=plsc.VectorSubcoreMesh(core_axis_name="c", subcore_axis_name="s"),
        scratch_types=[
            pltpu.VMEM((_VC, P), jnp.int32),
            pltpu.VMEM((_VC, P), jnp.float32),
            pltpu.SemaphoreType.DMA,
        ],
    )(_extract_body)


def _gather_body(x_hbm, idx_hbm, out_hbm, idx_v, re0, re1, ro0, ro1,
                 sp0, sp1, sem_g, sem_o, st_sem0, st_sem1):
    cid = lax.axis_index("c")
    sid = lax.axis_index("s")
    pltpu.sync_copy(idx_hbm.at[sid, pl.ds(cid * BPC, BPC)], idx_v)
    sps = [sp0, sp1]
    st_sems = [st_sem0, st_sem1]
    st_cps = [None, None]

    def start_stage(step):
        b = cid * BPC + step
        buf = step % 2
        st_cps[buf] = pltpu.async_copy(
            x_hbm.at[pl.ds(b * N + sid * SR, SR)],
            sps[buf].at[pl.ds(sid * SR, SR)],
            st_sems[buf])

    start_stage(0)
    start_stage(1)
    rows_e = [re0, re1]
    rows_o = [ro0, ro1]
    out_cps = [None, None]
    for step in range(BPC):
        buf = step % 2
        st_cps[buf].wait()
        if out_cps[buf] is not None:
            out_cps[buf][0].wait()
            out_cps[buf][1].wait()
        plsc.subcore_barrier()
        cp_e = pltpu.async_copy(sps[buf].at[idx_v.at[step, 0]], rows_e[buf],
                                sem_g)
        cp_o = pltpu.async_copy(sps[buf].at[idx_v.at[step, 1]], rows_o[buf],
                                sem_g)
        cp_e.wait()
        cp_o.wait()
        b = cid * BPC + step
        gbase = b * OG + sid * GH
        out_cps[buf] = (
            pltpu.async_copy(rows_e[buf],
                             out_hbm.at[pl.ds(gbase, GH), pl.ds(0, P)], sem_o),
            pltpu.async_copy(rows_o[buf],
                             out_hbm.at[pl.ds(gbase, GH), pl.ds(P, P)], sem_o),
        )
        plsc.subcore_barrier()
        if step + 2 < BPC:
            start_stage(step + 2)
    for cps in out_cps:
        cps[0].wait()
        cps[1].wait()


@functools.lru_cache(maxsize=None)
def _build_gather():
    return functools.partial(
        pl.kernel,
        out_type=jax.ShapeDtypeStruct((TOT // C, C * P), jnp.float32),
        mesh=plsc.VectorSubcoreMesh(core_axis_name="c", subcore_axis_name="s"),
        scratch_types=[
            pltpu.VMEM((BPC, 2, GH), jnp.int32),
            pltpu.VMEM((GH, P), jnp.float32),
            pltpu.VMEM((GH, P), jnp.float32),
            pltpu.VMEM((GH, P), jnp.float32),
            pltpu.VMEM((GH, P), jnp.float32),
            pltpu.VMEM_SHARED((N, P), jnp.float32),
            pltpu.VMEM_SHARED((N, P), jnp.float32),
            pltpu.SemaphoreType.DMA,
            pltpu.SemaphoreType.DMA,
            pltpu.SemaphoreType.DMA,
            pltpu.SemaphoreType.DMA,
        ],
    )(_gather_body)


def _mm_body(xs_ref, w_ref, o_ref):
    o_ref[...] = jnp.dot(xs_ref[...], w_ref[...],
                         preferred_element_type=jnp.float32)


_mm = pl.pallas_call(
    _mm_body,
    grid=(TOT // C // MB,),
    in_specs=[
        pl.BlockSpec((MB, P * C), lambda i: (i, 0)),
        pl.BlockSpec((P * C, P), lambda i: (0, 0)),
    ],
    out_specs=pl.BlockSpec((MB, P), lambda i: (i, 0)),
    out_shape=jax.ShapeDtypeStruct((TOT // C, P), jnp.float32),
)


def kernel(x, trafo):
    x2d = x.reshape(B * N, P)
    eidx = (jnp.arange(B * N, dtype=jnp.int32) * P + (P - 1)).reshape(
        B, _VC, P)
    vals = _build_extract()(x.reshape(-1), eidx)
    values_f = vals.reshape(B, 2, NH).transpose(2, 1, 0).reshape(NH, 2 * B)
    jjs, kks = _stage_params()
    order_t = _topk_sort(jjs, kks, values_f)
    order = order_t.T.reshape(B, NS, GH, 2).transpose(1, 0, 3, 2)
    xs = _build_gather()(x2d, order)
    traf = _mm(xs, trafo)
    return traf.reshape(B, OG, P)
